# baseline (speedup 1.0000x reference)
import jax
import jax.numpy as jnp
from jax import lax
from jax.experimental import pallas as pl
from jax.experimental.pallas import tpu as pltpu

_DIM = 512
_HID = 512
_E = 16
_LIMIT = 7.0
_TOK = 64
_PC = 128
_EPB = 4


def _routing_coef(x, gw, gb):
    g = jnp.dot(x, gw, preferred_element_type=jnp.float32) + gb
    ids = lax.broadcasted_iota(jnp.int32, (_TOK, _E), 1)
    m1 = jnp.max(g, axis=1, keepdims=True)
    i1 = jnp.min(jnp.where(g == m1, ids, _E), axis=1, keepdims=True)
    sel1 = ids == i1
    gm = jnp.where(sel1, -jnp.inf, g)
    m2 = jnp.max(gm, axis=1, keepdims=True)
    i2 = jnp.min(jnp.where(gm == m2, ids, _E), axis=1, keepdims=True)
    sel2 = ids == i2
    w2nd = jnp.exp(m2 - m1)
    denom = 1.0 + w2nd
    return (sel1.astype(jnp.float32) + sel2.astype(jnp.float32) * w2nd) / denom


def _moe_body(x_ref, gw_ref, gb_ref, w1_ref, b1_ref, w2_ref, b2_ref, out_ref,
              pe_ref, coef_ref):
    step = pl.program_id(0)
    x = x_ref[...]

    @pl.when(step == 0)
    def _():
        rows = lax.broadcasted_iota(jnp.int32, (2 * _PC, _PC), 0)
        cols = lax.broadcasted_iota(jnp.int32, (2 * _PC, _PC), 1)
        pe_ref[...] = (rows == 2 * cols).astype(jnp.float32)
        coef_ref[...] = _routing_coef(x, gw_ref[...], gb_ref[...])

    ids = lax.broadcasted_iota(jnp.int32, (_TOK, _E), 1)
    coef = coef_ref[...]

    pairs = []
    for ex in range(_EPB):
        h = lax.dot_general(x, w1_ref[ex], (((1,), (1,)), ((), ())),
                            preferred_element_type=jnp.float32)
        h = h + b1_ref[ex]
        hs = pltpu.roll(h, 2 * _HID - 1, axis=1)
        hg = jnp.minimum(h, _LIMIT)
        pairs.append(hg * jax.nn.sigmoid(1.702 * hg)
                     * (jnp.clip(hs, -_LIMIT, _LIMIT) + 1.0))
    pair = jnp.concatenate(pairs, axis=0)

    pe = pe_ref[...]
    act = jnp.concatenate(
        [lax.dot_general(pair[:, 2 * _PC * c:2 * _PC * (c + 1)], pe,
                         (((1,), (0,)), ((), ())),
                         preferred_element_type=jnp.float32)
         for c in range(_HID // _PC)], axis=1)

    contrib = None
    for ex in range(_EPB):
        e = _EPB * step + ex
        ce = jnp.sum(jnp.where(ids == e, coef, 0.0), axis=1, keepdims=True)
        y = lax.dot_general(act[_TOK * ex:_TOK * (ex + 1)], w2_ref[ex],
                            (((1,), (1,)), ((), ())),
                            preferred_element_type=jnp.float32)
        y = y + b2_ref[ex]
        contrib = ce * y if contrib is None else contrib + ce * y

    @pl.when(step == 0)
    def _():
        out_ref[...] = contrib

    @pl.when(step > 0)
    def _():
        out_ref[...] = out_ref[...] + contrib


def kernel(x, gate_w, gate_b, w1, b1, w2, b2):
    return pl.pallas_call(
        _moe_body,
        grid=(_E // _EPB,),
        in_specs=[
            pl.BlockSpec((_TOK, _DIM), lambda e: (0, 0)),
            pl.BlockSpec((_DIM, _E), lambda e: (0, 0)),
            pl.BlockSpec((1, _E), lambda e: (0, 0)),
            pl.BlockSpec((_EPB, 2 * _HID, _DIM), lambda e: (e, 0, 0)),
            pl.BlockSpec((_EPB, 1, 2 * _HID), lambda e: (e, 0, 0)),
            pl.BlockSpec((_EPB, _DIM, _HID), lambda e: (e, 0, 0)),
            pl.BlockSpec((_EPB, 1, _DIM), lambda e: (e, 0, 0)),
        ],
        out_specs=pl.BlockSpec((_TOK, _DIM), lambda e: (0, 0)),
        out_shape=jax.ShapeDtypeStruct((_TOK, _DIM), jnp.float32),
        scratch_shapes=[pltpu.VMEM((2 * _PC, _PC), jnp.float32),
                        pltpu.VMEM((_TOK, _E), jnp.float32)],
        compiler_params=pltpu.CompilerParams(
            dimension_semantics=("arbitrary",),
        ),
    )(x, gate_w, jnp.reshape(gate_b, (1, _E)), w1,
      jnp.reshape(b1, (_E, 1, 2 * _HID)), w2, jnp.reshape(b2, (_E, 1, _DIM)))


# speedup vs baseline: 1.2185x; 1.2185x over previous
import jax
import jax.numpy as jnp
from jax import lax
from jax.experimental import pallas as pl
from jax.experimental.pallas import tpu as pltpu

_DIM = 512
_HID = 512
_E = 16
_LIMIT = 7.0
_TOK = 64
_PC = 128
_EPB = 2
_STEPS = _E // _EPB


def _routing_coef(x, gw, gb):
    g = jnp.dot(x, gw, preferred_element_type=jnp.float32) + gb
    ids = lax.broadcasted_iota(jnp.int32, (_TOK, _E), 1)
    m1 = jnp.max(g, axis=1, keepdims=True)
    i1 = jnp.min(jnp.where(g == m1, ids, _E), axis=1, keepdims=True)
    sel1 = ids == i1
    gm = jnp.where(sel1, -jnp.inf, g)
    m2 = jnp.max(gm, axis=1, keepdims=True)
    i2 = jnp.min(jnp.where(gm == m2, ids, _E), axis=1, keepdims=True)
    sel2 = ids == i2
    w2nd = jnp.exp(m2 - m1)
    denom = 1.0 + w2nd
    return (sel1.astype(jnp.float32) + sel2.astype(jnp.float32) * w2nd) / denom


def _moe_body(x_ref, gw_ref, gb_ref, w1_hbm, b1_ref, w2_hbm, b2_ref, out_ref,
              w1buf, w2buf, pe_ref, coef_ref, sem):
    step = pl.program_id(0)
    x = x_ref[...]

    def w1_copy(blk, slot):
        return pltpu.make_async_copy(
            w1_hbm.at[pl.ds(blk * _EPB, _EPB)], w1buf.at[slot], sem.at[slot])

    def w2_copy(blk, slot):
        return pltpu.make_async_copy(
            w2_hbm.at[pl.ds(blk * _EPB, _EPB)], w2buf.at[slot], sem.at[slot])

    @pl.when(step == 0)
    def _():
        w1_copy(0, 0).start()
        w2_copy(0, 0).start()
        w1_copy(1, 1).start()
        w2_copy(1, 1).start()
        rows = lax.broadcasted_iota(jnp.int32, (2 * _PC, _PC), 0)
        cols = lax.broadcasted_iota(jnp.int32, (2 * _PC, _PC), 1)
        pe_ref[...] = (rows == 2 * cols).astype(jnp.float32)
        coef_ref[...] = _routing_coef(x, gw_ref[...], gb_ref[...])

    slot = lax.rem(step, 2)

    @pl.when(jnp.logical_and(step >= 1, step < _STEPS - 1))
    def _():
        nxt = lax.rem(step + 1, 2)
        w1_copy(step + 1, nxt).start()
        w2_copy(step + 1, nxt).start()

    w1_copy(step, slot).wait()
    w2_copy(step, slot).wait()

    ids = lax.broadcasted_iota(jnp.int32, (_TOK, _E), 1)
    coef = coef_ref[...]

    pairs = []
    for ex in range(_EPB):
        h = lax.dot_general(x, w1buf[slot, ex], (((1,), (1,)), ((), ())),
                            preferred_element_type=jnp.float32)
        h = h + b1_ref[0, _EPB * step + ex]
        hs = pltpu.roll(h, 2 * _HID - 1, axis=1)
        hg = jnp.minimum(h, _LIMIT)
        pairs.append(hg * jax.nn.sigmoid(1.702 * hg)
                     * (jnp.clip(hs, -_LIMIT, _LIMIT) + 1.0))
    pair = jnp.concatenate(pairs, axis=0)

    pe = pe_ref[...]
    act = jnp.concatenate(
        [lax.dot_general(pair[:, 2 * _PC * c:2 * _PC * (c + 1)], pe,
                         (((1,), (0,)), ((), ())),
                         preferred_element_type=jnp.float32)
         for c in range(_HID // _PC)], axis=1)

    contrib = None
    for ex in range(_EPB):
        e = _EPB * step + ex
        ce = jnp.sum(jnp.where(ids == e, coef, 0.0), axis=1, keepdims=True)
        y = lax.dot_general(act[_TOK * ex:_TOK * (ex + 1)], w2buf[slot, ex],
                            (((1,), (1,)), ((), ())),
                            preferred_element_type=jnp.float32)
        y = y + b2_ref[0, e]
        contrib = ce * y if contrib is None else contrib + ce * y

    @pl.when(step == 0)
    def _():
        out_ref[...] = contrib

    @pl.when(step > 0)
    def _():
        out_ref[...] = out_ref[...] + contrib


def kernel(x, gate_w, gate_b, w1, b1, w2, b2):
    return pl.pallas_call(
        _moe_body,
        grid=(_STEPS,),
        in_specs=[
            pl.BlockSpec((_TOK, _DIM), lambda s: (0, 0)),
            pl.BlockSpec((_DIM, _E), lambda s: (0, 0)),
            pl.BlockSpec((1, _E), lambda s: (0, 0)),
            pl.BlockSpec(memory_space=pl.ANY),
            pl.BlockSpec((1, _E, 2 * _HID), lambda s: (0, 0, 0)),
            pl.BlockSpec(memory_space=pl.ANY),
            pl.BlockSpec((1, _E, _DIM), lambda s: (0, 0, 0)),
        ],
        out_specs=pl.BlockSpec((_TOK, _DIM), lambda s: (0, 0)),
        out_shape=jax.ShapeDtypeStruct((_TOK, _DIM), jnp.float32),
        scratch_shapes=[
            pltpu.VMEM((2, _EPB, 2 * _HID, _DIM), jnp.float32),
            pltpu.VMEM((2, _EPB, _DIM, _HID), jnp.float32),
            pltpu.VMEM((2 * _PC, _PC), jnp.float32),
            pltpu.VMEM((_TOK, _E), jnp.float32),
            pltpu.SemaphoreType.DMA((2,)),
        ],
        compiler_params=pltpu.CompilerParams(
            dimension_semantics=("arbitrary",),
        ),
    )(x, gate_w, jnp.reshape(gate_b, (1, _E)), w1,
      jnp.reshape(b1, (1, _E, 2 * _HID)), w2, jnp.reshape(b2, (1, _E, _DIM)))
